# Initial kernel scaffold; baseline (speedup 1.0000x reference)
import jax
import jax.numpy as jnp
from jax.experimental import pallas as pl


def kernel(x, edge_index, W1, b1, W2, b2, Wc, bc):
    raise NotImplementedError("write your pallas kernel here")


# speedup vs baseline: 14.5020x; 14.5020x over previous
import functools

import jax
import jax.numpy as jnp
from jax import lax
from jax.experimental import pallas as pl
from jax.experimental.pallas import tpu as pltpu
from jax.experimental.pallas import tpu_sc as plsc

N = 10000
NPAD = 10240
E = 320000
D = 128
C = 16

NC = 2
NS = 16
NW = NC * NS
EPW = E // NW
CHUNK = 128
NFULL = EPW // CHUNK
REM = EPW - NFULL * CHUNK
RPT = NPAD // NS
ZROWS = 128

_mesh = plsc.VectorSubcoreMesh(core_axis_name="c", subcore_axis_name="s")


@functools.partial(
    pl.kernel,
    out_type=jax.ShapeDtypeStruct((NC, NPAD, 16), jnp.float32),
    mesh=_mesh,
    scratch_types=[
        pltpu.VMEM_SHARED((NPAD, 16), jnp.float32),
        pltpu.VMEM((CHUNK,), jnp.int32),
        pltpu.VMEM((REM,), jnp.int32),
        pltpu.VMEM((CHUNK, 16), jnp.float32),
        pltpu.VMEM((RPT, 16), jnp.float32),
    ],
)
def _deg_kernel(dst_hbm, out_hbm, deg_sh, didx, didx_r, ones_v, stage):
    cid = lax.axis_index("c")
    sid = lax.axis_index("s")
    wid = sid * NC + cid

    def fill(buf, rows, val):
        def fo(i, _):
            buf[i, pl.ds(0, 16)] = jnp.full((16,), val, jnp.float32)
            return 0
        lax.fori_loop(0, rows, fo, 0)

    fill(ones_v, CHUNK, 1.0)
    fill(stage, RPT, 0.0)

    r0 = sid * RPT
    pltpu.sync_copy(stage, deg_sh.at[pl.ds(r0, RPT)])
    plsc.subcore_barrier()

    base = wid * EPW

    def body(i, _):
        pltpu.sync_copy(dst_hbm.at[pl.ds(base + i * CHUNK, CHUNK)], didx)
        pltpu.sync_copy(ones_v, deg_sh.at[didx], add=True)
        return 0

    lax.fori_loop(0, NFULL, body, 0)
    pltpu.sync_copy(dst_hbm.at[pl.ds(base + NFULL * CHUNK, REM)], didx_r)
    pltpu.sync_copy(ones_v.at[pl.ds(0, REM)], deg_sh.at[didx_r], add=True)

    plsc.subcore_barrier()
    pltpu.sync_copy(deg_sh.at[pl.ds(r0, RPT)], stage)
    pltpu.sync_copy(stage, out_hbm.at[cid, pl.ds(r0, RPT)])


@functools.partial(
    pl.kernel,
    out_type=jax.ShapeDtypeStruct((NC, NPAD, D), jnp.float32),
    mesh=_mesh,
    scratch_types=[
        pltpu.VMEM_SHARED((NPAD, D), jnp.float32),
        pltpu.VMEM((CHUNK,), jnp.int32),
        pltpu.VMEM((CHUNK,), jnp.int32),
        pltpu.VMEM((CHUNK, D), jnp.float32),
        pltpu.VMEM((REM,), jnp.int32),
        pltpu.VMEM((REM,), jnp.int32),
        pltpu.VMEM((REM, D), jnp.float32),
        pltpu.VMEM((ZROWS, D), jnp.float32),
        pltpu.SemaphoreType.DMA,
    ],
)
def _agg_kernel(hs_hbm, src_hbm, dst_hbm, out_hbm, acc_sh,
                sidx, didx, rows, sidx_r, didx_r, rows_r, zbuf, sem):
    cid = lax.axis_index("c")
    sid = lax.axis_index("s")
    wid = sid * NC + cid

    def zfill(i, _):
        def zf2(j, _):
            zbuf[i, pl.ds(j * 16, 16)] = jnp.zeros((16,), jnp.float32)
            return 0
        lax.fori_loop(0, D // 16, zf2, 0)
        return 0

    lax.fori_loop(0, ZROWS, zfill, 0)

    r0 = sid * RPT
    for k in range(RPT // ZROWS):
        pltpu.sync_copy(zbuf, acc_sh.at[pl.ds(r0 + k * ZROWS, ZROWS)])
    plsc.subcore_barrier()

    base = wid * EPW

    def body(i, _):
        off = base + i * CHUNK
        pltpu.sync_copy(src_hbm.at[pl.ds(off, CHUNK)], sidx)
        pltpu.sync_copy(dst_hbm.at[pl.ds(off, CHUNK)], didx)
        pltpu.async_copy(hs_hbm.at[sidx], rows, sem).wait()
        pltpu.sync_copy(rows, acc_sh.at[didx], add=True)
        return 0

    lax.fori_loop(0, NFULL, body, 0)

    off = base + NFULL * CHUNK
    pltpu.sync_copy(src_hbm.at[pl.ds(off, REM)], sidx_r)
    pltpu.sync_copy(dst_hbm.at[pl.ds(off, REM)], didx_r)
    pltpu.async_copy(hs_hbm.at[sidx_r], rows_r, sem).wait()
    pltpu.sync_copy(rows_r, acc_sh.at[didx_r], add=True)

    plsc.subcore_barrier()
    for k in range(RPT // ZROWS):
        pltpu.sync_copy(acc_sh.at[pl.ds(r0 + k * ZROWS, ZROWS)], zbuf)
        pltpu.sync_copy(zbuf, out_hbm.at[cid, pl.ds(r0 + k * ZROWS, ZROWS)])


BN = 1024
GRID = (N + BN - 1) // BN


def _mm_body(x_ref, w_ref, o_ref):
    o_ref[...] = jnp.dot(x_ref[...], w_ref[...],
                         preferred_element_type=jnp.float32)


_h1p_call = pl.pallas_call(
    _mm_body,
    grid=(GRID,),
    in_specs=[pl.BlockSpec((BN, D), lambda i: (i, 0)),
              pl.BlockSpec((D, D), lambda i: (0, 0))],
    out_specs=pl.BlockSpec((BN, D), lambda i: (i, 0)),
    out_shape=jax.ShapeDtypeStruct((N, D), jnp.float32),
)


def _tc2_body(h1p_ref, degs_ref, hs1_ref, dis_ref):
    deg = degs_ref[0, :, 0:1] + degs_ref[1, :, 0:1] + 1.0
    dis = lax.rsqrt(deg)
    dis_ref[...] = dis
    hs1_ref[...] = h1p_ref[...] * dis


_tc2_call = pl.pallas_call(
    _tc2_body,
    grid=(GRID,),
    in_specs=[pl.BlockSpec((BN, D), lambda i: (i, 0)),
              pl.BlockSpec((NC, BN, 16), lambda i: (0, i, 0))],
    out_specs=[pl.BlockSpec((BN, D), lambda i: (i, 0)),
               pl.BlockSpec((BN, 1), lambda i: (i, 0))],
    out_shape=[jax.ShapeDtypeStruct((N, D), jnp.float32),
               jax.ShapeDtypeStruct((N, 1), jnp.float32)],
)


def _tc3_body(p_ref, hs1_ref, dis_ref, b1_ref, w2_ref, h1_ref, hs2_ref):
    dis = dis_ref[...]
    agg = p_ref[0] + p_ref[1] + hs1_ref[...]
    h1 = jnp.maximum(dis * agg + b1_ref[...], 0.0)
    h1_ref[...] = h1
    hs2_ref[...] = jnp.dot(h1, w2_ref[...],
                           preferred_element_type=jnp.float32) * dis


_tc3_call = pl.pallas_call(
    _tc3_body,
    grid=(GRID,),
    in_specs=[pl.BlockSpec((NC, BN, D), lambda i: (0, i, 0)),
              pl.BlockSpec((BN, D), lambda i: (i, 0)),
              pl.BlockSpec((BN, 1), lambda i: (i, 0)),
              pl.BlockSpec((1, D), lambda i: (0, 0)),
              pl.BlockSpec((D, D), lambda i: (0, 0))],
    out_specs=[pl.BlockSpec((BN, D), lambda i: (i, 0)),
               pl.BlockSpec((BN, D), lambda i: (i, 0))],
    out_shape=[jax.ShapeDtypeStruct((N, D), jnp.float32),
               jax.ShapeDtypeStruct((N, D), jnp.float32)],
)


def _tc4_body(q_ref, hs2_ref, dis_ref, b2_ref, wc_ref, bc_ref,
              h2_ref, cls_ref):
    dis = dis_ref[...]
    agg = q_ref[0] + q_ref[1] + hs2_ref[...]
    h2 = jnp.maximum(dis * agg + b2_ref[...], 0.0)
    h2_ref[...] = h2
    cls_ref[...] = jnp.dot(h2, wc_ref[...],
                           preferred_element_type=jnp.float32) + bc_ref[...]


_tc4_call = pl.pallas_call(
    _tc4_body,
    grid=(GRID,),
    in_specs=[pl.BlockSpec((NC, BN, D), lambda i: (0, i, 0)),
              pl.BlockSpec((BN, D), lambda i: (i, 0)),
              pl.BlockSpec((BN, 1), lambda i: (i, 0)),
              pl.BlockSpec((1, D), lambda i: (0, 0)),
              pl.BlockSpec((D, C), lambda i: (0, 0)),
              pl.BlockSpec((1, C), lambda i: (0, 0))],
    out_specs=[pl.BlockSpec((BN, D), lambda i: (i, 0)),
               pl.BlockSpec((BN, C), lambda i: (i, 0))],
    out_shape=[jax.ShapeDtypeStruct((N, D), jnp.float32),
               jax.ShapeDtypeStruct((N, C), jnp.float32)],
)


def kernel(x, edge_index, W1, b1, W2, b2, Wc, bc):
    src = edge_index[0]
    dst = edge_index[1]

    degs = _deg_kernel(dst)
    h1p = _h1p_call(x, W1)
    hs1, dis = _tc2_call(h1p, degs)

    p = _agg_kernel(hs1, src, dst)
    h1, hs2 = _tc3_call(p, hs1, dis, b1.reshape(1, D), W2)

    q = _agg_kernel(hs2, src, dst)
    h2, cls = _tc4_call(q, hs2, dis, b2.reshape(1, D), Wc, bc.reshape(1, C))

    feat_cat = jnp.concatenate([h1, h2, cls], axis=1)
    return (cls, feat_cat)
